# baseline (speedup 1.0000x reference)
import functools

import jax
import jax.numpy as jnp
from jax import lax
from jax.experimental import pallas as pl
from jax.experimental.pallas import tpu as pltpu
from jax.experimental.pallas import tpu_sc as plsc

_NUM_CORES = 2
_NUM_SUBCORES = 16
_LANES = 16
_N_CHUNKS = 8
_N_BGROUPS = 4


def _make_sc_kernel(B, N, H):
    nodes_per_chunk = N // _N_CHUNKS
    b_per_group = B // _N_BGROUPS
    chunks_per_row = H // _LANES

    mesh = plsc.VectorSubcoreMesh(
        core_axis_name="c", subcore_axis_name="s")

    @functools.partial(
        pl.kernel,
        out_type=jax.ShapeDtypeStruct((B, N, H), jnp.float32),
        mesh=mesh,
        scratch_types=[
            pltpu.VMEM((nodes_per_chunk,), jnp.int32),
            pltpu.VMEM((nodes_per_chunk,), jnp.int32),
            pltpu.VMEM((nodes_per_chunk, H), jnp.float32),
            pltpu.VMEM((nodes_per_chunk, H), jnp.float32),
            pltpu.SemaphoreType.DMA,
            pltpu.SemaphoreType.DMA,
        ],
    )
    def sc_kernel(in_deg, out_deg, w_in, w_out, out,
                  idx_in_v, idx_out_v, a_v, b_v, gsem, wsem):
        c = lax.axis_index("c")
        s = lax.axis_index("s")
        wid = s * _NUM_CORES + c
        chunk = wid % _N_CHUNKS
        bgroup = wid // _N_CHUNKS
        node0 = chunk * nodes_per_chunk
        b0 = bgroup * b_per_group

        pltpu.sync_copy(in_deg.at[pl.ds(node0, nodes_per_chunk)], idx_in_v)
        pltpu.sync_copy(out_deg.at[pl.ds(node0, nodes_per_chunk)], idx_out_v)

        cp_a = pltpu.async_copy(w_in.at[idx_in_v], a_v, gsem)
        cp_b = pltpu.async_copy(w_out.at[idx_out_v], b_v, gsem)
        cp_a.wait()
        cp_b.wait()

        for j in range(nodes_per_chunk):
            def add_body(k, _, j=j):
                sl = pl.ds(k * _LANES, _LANES)
                a_v[j, sl] = a_v[j, sl] + b_v[j, sl]
                return _
            lax.fori_loop(0, chunks_per_row, add_body, None)

        copies = []
        for i in range(b_per_group):
            copies.append(
                pltpu.async_copy(
                    a_v, out.at[b0 + i, pl.ds(node0, nodes_per_chunk)], wsem))
        for cp in copies:
            cp.wait()

    return sc_kernel


@jax.jit
def kernel(x, in_degree, out_degree, W_in, W_out):
    B = x.shape[0]
    N = in_degree.shape[0]
    H = W_in.shape[1]
    sc = _make_sc_kernel(B, N, H)
    return sc(in_degree, out_degree, W_in, W_out)
